# baseline (speedup 1.0000x reference)
import functools

import jax
import jax.numpy as jnp
import numpy as np
from jax import lax
from jax.experimental import pallas as pl
from jax.experimental.pallas import tpu as pltpu
from jax.experimental.pallas import tpu_sc as plsc

_N = 10000
_E = 320000
_BLK_N = 5000
_BLK_E = 8000
_C = 128
_NW = 32
_EPT = _E // _NW
_NFULL = _EPT // _C
_TAIL = _EPT - _NFULL * _C
_TPW = _N // 16

_f32 = jnp.float32
_bf16 = jnp.bfloat16
_u32 = jnp.uint32
_MESH = plsc.VectorSubcoreMesh(core_axis_name="c", subcore_axis_name="s")
_SC_PARAMS = pltpu.CompilerParams(use_tc_tiling_on_sc=False)


def _sc_gather_body(ps, pd, src_i, dst_i, gs, gd,
                    isv, idv, rs0, rd0, rs1, rd1, rs2, rd2,
                    sg0, sg1, sg2, so0, so1, so2):
    wid = lax.axis_index("s") * 2 + lax.axis_index("c")
    e0 = wid * _EPT
    pltpu.sync_copy(src_i.at[pl.ds(e0, _EPT)], isv)
    pltpu.sync_copy(dst_i.at[pl.ds(e0, _EPT)], idv)

    rsb = (rs0, rs1, rs2)
    rdb = (rd0, rd1, rd2)
    sg = (sg0, sg1, sg2)
    so = (so0, so1, so2)

    def issue(c, s):
        sl = pl.ds(c * _C, _C)
        pltpu.async_copy(ps.at[isv.at[sl]], rsb[s], sg[s])
        pltpu.async_copy(pd.at[idv.at[sl]], rdb[s], sg[s])

    def waitg(s):
        pltpu.make_async_copy(ps.at[isv.at[pl.ds(0, _C)]], rsb[s],
                              sg[s]).wait()
        pltpu.make_async_copy(pd.at[idv.at[pl.ds(0, _C)]], rdb[s],
                              sg[s]).wait()

    def store(c, s):
        sl = pl.ds(e0 + c * _C, _C)
        pltpu.async_copy(rsb[s], gs.at[sl], so[s])
        pltpu.async_copy(rdb[s], gd.at[sl], so[s])

    def drain(s):
        sl = pl.ds(e0, _C)
        pltpu.make_async_copy(rsb[s], gs.at[sl], so[s]).wait()
        pltpu.make_async_copy(rdb[s], gd.at[sl], so[s]).wait()

    for s in range(3):
        issue(s, s)

    def trip(t, carry):
        c = 3 * t
        for s in range(3):
            waitg(s)
            store(c + s, s)
        for s in range(3):
            nxt = c + 3 + s

            @pl.when(nxt < _NFULL)
            def _():
                drain(s)
                issue(nxt, s)

        return carry

    lax.fori_loop(0, _NFULL // 3, trip, 0)
    for s in range(3):
        drain(s)

    tsl = pl.ds(_NFULL * _C, _TAIL)
    rs_t = rs0.at[pl.ds(0, _TAIL)]
    rd_t = rd0.at[pl.ds(0, _TAIL)]
    pltpu.async_copy(ps.at[isv.at[tsl]], rs_t, sg0)
    pltpu.async_copy(pd.at[idv.at[tsl]], rd_t, sg0)
    pltpu.make_async_copy(ps.at[isv.at[tsl]], rs_t, sg0).wait()
    pltpu.make_async_copy(pd.at[idv.at[tsl]], rd_t, sg0).wait()
    osl = pl.ds(e0 + _NFULL * _C, _TAIL)
    pltpu.sync_copy(rs_t, gs.at[osl])
    pltpu.sync_copy(rd_t, gd.at[osl])


_sc_gather = pl.kernel(
    _sc_gather_body,
    out_type=[jax.ShapeDtypeStruct((_E, 32), _u32)] * 2,
    mesh=_MESH,
    scratch_types=[
        pltpu.VMEM((_EPT,), jnp.int32),
        pltpu.VMEM((_EPT,), jnp.int32),
        pltpu.VMEM((_C, 32), _u32),
        pltpu.VMEM((_C, 32), _u32),
        pltpu.VMEM((_C, 32), _u32),
        pltpu.VMEM((_C, 32), _u32),
        pltpu.VMEM((_C, 32), _u32),
        pltpu.VMEM((_C, 32), _u32),
        pltpu.SemaphoreType.DMA,
        pltpu.SemaphoreType.DMA,
        pltpu.SemaphoreType.DMA,
        pltpu.SemaphoreType.DMA,
        pltpu.SemaphoreType.DMA,
        pltpu.SemaphoreType.DMA,
    ],
    compiler_params=_SC_PARAMS,
)


def _sc_scatter_body(vals, dst_i, zz, out,
                     idv, v0, v1, v2, acc,
                     sv0, sv1, sv2, sa0, sa1, sa2):
    cid = lax.axis_index("c")
    sid = lax.axis_index("s")
    wid = sid * 2 + cid
    e0 = wid * _EPT
    r0 = sid * _TPW
    pltpu.sync_copy(zz.at[pl.ds(r0, _TPW)], acc.at[pl.ds(r0, _TPW)])
    pltpu.sync_copy(dst_i.at[pl.ds(e0, _EPT)], idv)
    plsc.subcore_barrier()

    vb = (v0, v1, v2)
    sv = (sv0, sv1, sv2)
    sa = (sa0, sa1, sa2)

    def load(c, s):
        pltpu.async_copy(vals.at[pl.ds(e0 + c * _C, _C)], vb[s], sv[s])

    def wait_load(s):
        pltpu.make_async_copy(vals.at[pl.ds(e0, _C)], vb[s], sv[s]).wait()

    def scat(c, s):
        pltpu.async_copy(vb[s], acc.at[idv.at[pl.ds(c * _C, _C)]], sa[s],
                         add=True)

    def wait_scat(s):
        pltpu.make_async_copy(vb[s], acc.at[idv.at[pl.ds(0, _C)]],
                              sa[s]).wait()

    for s in range(3):
        load(s, s)

    def trip(t, carry):
        c = 3 * t
        for s in range(3):
            wait_load(s)
            scat(c + s, s)
        for s in range(3):
            nxt = c + 3 + s

            @pl.when(nxt < _NFULL)
            def _():
                wait_scat(s)
                load(nxt, s)

        return carry

    lax.fori_loop(0, _NFULL // 3, trip, 0)
    for s in range(3):
        wait_scat(s)

    tv = v0.at[pl.ds(0, _TAIL)]
    pltpu.sync_copy(vals.at[pl.ds(e0 + _NFULL * _C, _TAIL)], tv)
    pltpu.sync_copy(tv, acc.at[idv.at[pl.ds(_NFULL * _C, _TAIL)]], add=True)

    plsc.subcore_barrier()
    pltpu.sync_copy(acc.at[pl.ds(r0, _TPW)], out.at[cid, pl.ds(r0, _TPW)])


_sc_scatter = pl.kernel(
    _sc_scatter_body,
    out_type=jax.ShapeDtypeStruct((2, _N, 32), _f32),
    mesh=_MESH,
    scratch_types=[
        pltpu.VMEM((_EPT,), jnp.int32),
        pltpu.VMEM((_C, 32), _f32),
        pltpu.VMEM((_C, 32), _f32),
        pltpu.VMEM((_C, 32), _f32),
        pltpu.VMEM_SHARED((_N, 32), _f32),
        pltpu.SemaphoreType.DMA,
        pltpu.SemaphoreType.DMA,
        pltpu.SemaphoreType.DMA,
        pltpu.SemaphoreType.DMA,
        pltpu.SemaphoreType.DMA,
        pltpu.SemaphoreType.DMA,
    ],
    compiler_params=_SC_PARAMS,
)


def _dot(a, b):
    return jnp.dot(a, b, preferred_element_type=_f32)


def _pack2(lo, hi):
    lob = lax.bitcast_convert_type(lo, _u32)
    hib = lax.bitcast_convert_type(hi, _u32)
    lo16 = ((lob + _u32(0x8000)) >> 16) & _u32(0xFFFF)
    hi16 = (hib + _u32(0x8000)) & _u32(0xFFFF0000)
    return hi16 | lo16


def _unpack2(w):
    lo = lax.bitcast_convert_type(w << 16, _f32)
    hi = lax.bitcast_convert_type(w & _u32(0xFFFF0000), _f32)
    return jnp.concatenate([lo, hi], axis=1)


def _tc_nodeproj_body(x, wp, ps, pd):
    p = _dot(x[...], wp[...])
    ps[...] = _pack2(p[:, 0:32], p[:, 32:64])
    pd[...] = _pack2(p[:, 64:96], p[:, 96:128])


def _tc_edge1_body(gs, gd, ea, w1e, b1, w2, b2, out):
    h = jnp.maximum(_unpack2(gs[...]) + _unpack2(gd[...])
                    + _dot(ea[...], w1e[...]) + b1[...], 0.0)
    out[...] = _dot(h, w2[...]) + b2[...]


def _tc_node_mid_body(x, aggp, wx, wa, bih, bhh, wqx, wqh, h1o, qso, qdo):
    agg = aggp[0] + aggp[1]
    gi = _dot(x[...], wx[...]) + _dot(agg, wa[...]) + bih[...]
    b = bhh[...]
    r = jax.nn.sigmoid(gi[:, :64] + b[:, :64])
    z = jax.nn.sigmoid(gi[:, 64:128] + b[:, 64:128])
    nn_ = jnp.tanh(gi[:, 128:] + r * b[:, 128:])
    h1 = (1.0 - z) * nn_
    q = _dot(x[...], wqx[...]) + _dot(h1, wqh[...])
    h1o[...] = h1
    qso[...] = _pack2(q[:, 0:32], q[:, 32:64])
    qdo[...] = _pack2(q[:, 64:96], q[:, 96:128])


def _tc_edge2_body(gs, gd, ea, e1, wea, we1, b1, w2, b2, out):
    h = jnp.maximum(
        _unpack2(gs[...]) + _unpack2(gd[...]) + _dot(ea[...], wea[...])
        + _dot(e1[...], we1[...]) + b1[...], 0.0)
    out[...] = _dot(h, w2[...]) + b2[...]


def _tc_node_out_body(x, h1, aggp, wx, wh, wa, b1, w2, b2, out):
    agg = aggp[0] + aggp[1]
    h3 = jnp.maximum(
        _dot(x[...], wx[...]) + _dot(h1[...], wh[...]) + _dot(agg, wa[...])
        + b1[...], 0.0)
    out[...] = _dot(h3, w2[...]) + b2[...]


def _full(shape):
    return pl.BlockSpec(shape, lambda i: tuple(0 for _ in shape))


def _rows(blk, cols):
    return pl.BlockSpec((blk, cols), lambda i: (i, 0))


def _aggspec(blk):
    return pl.BlockSpec((2, blk, 32), lambda i: (0, i, 0))


_GRID_N = (_N // _BLK_N,)
_GRID_E = (_E // _BLK_E,)

_tc_nodeproj = pl.pallas_call(
    _tc_nodeproj_body,
    grid=_GRID_N,
    in_specs=[_rows(_BLK_N, 128), _full((128, 128))],
    out_specs=[_rows(_BLK_N, 32)] * 2,
    out_shape=[jax.ShapeDtypeStruct((_N, 32), _u32)] * 2,
)

_tc_edge1 = pl.pallas_call(
    _tc_edge1_body,
    grid=_GRID_E,
    in_specs=[_rows(_BLK_E, 32), _rows(_BLK_E, 32), _rows(_BLK_E, 16),
              _full((16, 64)), _full((1, 64)), _full((64, 32)), _full((1, 32))],
    out_specs=_rows(_BLK_E, 32),
    out_shape=jax.ShapeDtypeStruct((_E, 32), _f32),
)

_tc_node_mid = pl.pallas_call(
    _tc_node_mid_body,
    grid=_GRID_N,
    in_specs=[_rows(_BLK_N, 128), _aggspec(_BLK_N),
              _full((128, 192)), _full((32, 192)), _full((1, 192)),
              _full((1, 192)), _full((128, 128)), _full((64, 128))],
    out_specs=[_rows(_BLK_N, 64), _rows(_BLK_N, 32), _rows(_BLK_N, 32)],
    out_shape=[jax.ShapeDtypeStruct((_N, 64), _f32),
               jax.ShapeDtypeStruct((_N, 32), _u32),
               jax.ShapeDtypeStruct((_N, 32), _u32)],
)

_tc_edge2 = pl.pallas_call(
    _tc_edge2_body,
    grid=_GRID_E,
    in_specs=[_rows(_BLK_E, 32), _rows(_BLK_E, 32), _rows(_BLK_E, 16),
              _rows(_BLK_E, 32), _full((16, 64)), _full((32, 64)),
              _full((1, 64)), _full((64, 32)), _full((1, 32))],
    out_specs=_rows(_BLK_E, 32),
    out_shape=jax.ShapeDtypeStruct((_E, 32), _f32),
)

_tc_node_out = pl.pallas_call(
    _tc_node_out_body,
    grid=_GRID_N,
    in_specs=[_rows(_BLK_N, 128), _rows(_BLK_N, 64), _aggspec(_BLK_N),
              _full((128, 64)), _full((64, 64)), _full((32, 64)),
              _full((1, 64)), _full((64, 128)), _full((1, 128))],
    out_specs=_rows(_BLK_N, 128),
    out_shape=jax.ShapeDtypeStruct((_N, 128), _f32),
)


def kernel(x, edge_index, edge_attr, params):
    p = params
    src = edge_index[0]
    dst = edge_index[1]

    w1 = p["gn1_e_W1"]
    wp = jnp.concatenate([w1[:128], w1[128:256]], axis=1)
    w1e = w1[256:]
    b11 = p["gn1_e_b1"].reshape(1, 64)
    w12 = p["gn1_e_W2"]
    b12 = p["gn1_e_b2"].reshape(1, 32)

    wih_t = p["gru_Wih"].T
    wx = wih_t[:128]
    wa = wih_t[128:]
    bih = p["gru_bih"].reshape(1, 192)
    bhh = p["gru_bhh"].reshape(1, 192)

    w2_ = p["gn2_e_W1"]
    wq = jnp.concatenate([w2_[:192], w2_[192:384]], axis=1)
    wqx = wq[:128]
    wqh = wq[128:]
    wea2 = w2_[384:400]
    we12 = w2_[400:432]
    b21 = p["gn2_e_b1"].reshape(1, 64)
    w22 = p["gn2_e_W2"]
    b22 = p["gn2_e_b2"].reshape(1, 32)

    wn1 = p["gn2_n_W1"]
    wnx = wn1[:128]
    wnh = wn1[128:192]
    wna = wn1[192:]
    bn1 = p["gn2_n_b1"].reshape(1, 64)
    wn2 = p["gn2_n_W2"]
    bn2 = p["gn2_n_b2"].reshape(1, 128)

    zz = jnp.zeros((_N, 32), _f32)

    ps, pd = _tc_nodeproj(x, wp)
    g1s, g1d = _sc_gather(ps, pd, src, dst)
    e1 = _tc_edge1(g1s, g1d, edge_attr, w1e, b11, w12, b12)
    agg1p = _sc_scatter(e1, dst, zz)
    h1, qs, qd = _tc_node_mid(x, agg1p, wx, wa, bih, bhh, wqx, wqh)
    g2s, g2d = _sc_gather(qs, qd, src, dst)
    e2 = _tc_edge2(g2s, g2d, edge_attr, e1, wea2, we12, b21, w22, b22)
    agg2p = _sc_scatter(e2, dst, zz)
    out = _tc_node_out(x, h1, agg2p, wnx, wnh, wna, bn1, wn2, bn2)
    return out, h1[None]


# speedup vs baseline: 1.1008x; 1.1008x over previous
import functools

import jax
import jax.numpy as jnp
from jax import lax
from jax.experimental import pallas as pl
from jax.experimental.pallas import tpu as pltpu
from jax.experimental.pallas import tpu_sc as plsc

_N = 10000
_E = 320000
_BLK_N = 5000
_BLK_E = 8000
_C = 128
_NW = 32
_EPT = _E // _NW
_NFULL = _EPT // _C
_TAIL = _EPT - _NFULL * _C
_TPW = _N // 16

_f32 = jnp.float32
_MESH = plsc.VectorSubcoreMesh(core_axis_name="c", subcore_axis_name="s")
_SC_PARAMS = pltpu.CompilerParams(use_tc_tiling_on_sc=False)


def _add_rows(dst_v, a_v, b_v, nrows):
    def row(r, carry):
        for cc in range(4):
            sl = pl.ds(cc * 16, 16)
            dst_v[r, sl] = a_v[r, sl] + b_v[r, sl]
        return carry

    lax.fori_loop(0, nrows, row, 0)


def _sc_gather_body(ps, pd, src_i, dst_i, g,
                    isv, idv, rs0, rd0, rs1, rd1, g0, g1,
                    ss0, sd0, ss1, sd1, so0, so1):
    wid = lax.axis_index("s") * 2 + lax.axis_index("c")
    e0 = wid * _EPT
    pltpu.sync_copy(src_i.at[pl.ds(e0, _EPT)], isv)
    pltpu.sync_copy(dst_i.at[pl.ds(e0, _EPT)], idv)

    def issue(c, rs, rd, sem_s, sem_d):
        sl = pl.ds(c * _C, _C)
        pltpu.async_copy(ps.at[isv.at[sl]], rs, sem_s)
        pltpu.async_copy(pd.at[idv.at[sl]], rd, sem_d)

    def waitg(rs, rd, sem_s, sem_d):
        pltpu.make_async_copy(ps.at[isv.at[pl.ds(0, _C)]], rs, sem_s).wait()
        pltpu.make_async_copy(pd.at[idv.at[pl.ds(0, _C)]], rd, sem_d).wait()

    def store(c, gb, sem_o):
        pltpu.async_copy(gb, g.at[pl.ds(e0 + c * _C, _C)], sem_o)

    def drain(gb, sem_o):
        pltpu.make_async_copy(gb, g.at[pl.ds(e0, _C)], sem_o).wait()

    issue(0, rs0, rd0, ss0, sd0)

    def pair(t, carry):
        a = 2 * t
        issue(a + 1, rs1, rd1, ss1, sd1)
        waitg(rs0, rd0, ss0, sd0)

        @pl.when(t > 0)
        def _():
            drain(g0, so0)

        _add_rows(g0, rs0, rd0, _C)
        store(a, g0, so0)

        @pl.when(t < _NFULL // 2 - 1)
        def _():
            issue(a + 2, rs0, rd0, ss0, sd0)

        waitg(rs1, rd1, ss1, sd1)

        @pl.when(t > 0)
        def _():
            drain(g1, so1)

        _add_rows(g1, rs1, rd1, _C)
        store(a + 1, g1, so1)
        return carry

    lax.fori_loop(0, _NFULL // 2, pair, 0)
    drain(g0, so0)
    drain(g1, so1)

    tsl = pl.ds(_NFULL * _C, _TAIL)
    rs_t = rs0.at[pl.ds(0, _TAIL)]
    rd_t = rd0.at[pl.ds(0, _TAIL)]
    pltpu.async_copy(ps.at[isv.at[tsl]], rs_t, ss0)
    pltpu.async_copy(pd.at[idv.at[tsl]], rd_t, sd0)
    pltpu.make_async_copy(ps.at[isv.at[tsl]], rs_t, ss0).wait()
    pltpu.make_async_copy(pd.at[idv.at[tsl]], rd_t, sd0).wait()
    _add_rows(g0, rs0, rd0, _TAIL)
    pltpu.sync_copy(g0.at[pl.ds(0, _TAIL)],
                    g.at[pl.ds(e0 + _NFULL * _C, _TAIL)])


_sc_gather = pl.kernel(
    _sc_gather_body,
    out_type=jax.ShapeDtypeStruct((_E, 64), _f32),
    mesh=_MESH,
    scratch_types=[
        pltpu.VMEM((_EPT,), jnp.int32),
        pltpu.VMEM((_EPT,), jnp.int32),
        pltpu.VMEM((_C, 64), _f32),
        pltpu.VMEM((_C, 64), _f32),
        pltpu.VMEM((_C, 64), _f32),
        pltpu.VMEM((_C, 64), _f32),
        pltpu.VMEM((_C, 64), _f32),
        pltpu.VMEM((_C, 64), _f32),
        pltpu.SemaphoreType.DMA,
        pltpu.SemaphoreType.DMA,
        pltpu.SemaphoreType.DMA,
        pltpu.SemaphoreType.DMA,
        pltpu.SemaphoreType.DMA,
        pltpu.SemaphoreType.DMA,
    ],
    compiler_params=_SC_PARAMS,
)


def _sc_scatter_body(vals, dst_i, zz, out,
                     idv, v0, v1, v2, acc,
                     sv0, sv1, sv2, sa0, sa1, sa2):
    cid = lax.axis_index("c")
    sid = lax.axis_index("s")
    wid = sid * 2 + cid
    e0 = wid * _EPT
    r0 = sid * _TPW
    pltpu.sync_copy(zz.at[pl.ds(r0, _TPW)], acc.at[pl.ds(r0, _TPW)])
    pltpu.sync_copy(dst_i.at[pl.ds(e0, _EPT)], idv)
    plsc.subcore_barrier()

    vb = (v0, v1, v2)
    sv = (sv0, sv1, sv2)
    sa = (sa0, sa1, sa2)

    def load(c, s):
        pltpu.async_copy(vals.at[pl.ds(e0 + c * _C, _C)], vb[s], sv[s])

    def wait_load(s):
        pltpu.make_async_copy(vals.at[pl.ds(e0, _C)], vb[s], sv[s]).wait()

    def scat(c, s):
        pltpu.async_copy(vb[s], acc.at[idv.at[pl.ds(c * _C, _C)]], sa[s],
                         add=True)

    def wait_scat(s):
        pltpu.make_async_copy(vb[s], acc.at[idv.at[pl.ds(0, _C)]],
                              sa[s]).wait()

    for s in range(3):
        load(s, s)

    def trip(t, carry):
        c = 3 * t
        for s in range(3):
            wait_load(s)
            scat(c + s, s)
        for s in range(3):
            nxt = c + 3 + s

            @pl.when(nxt < _NFULL)
            def _():
                wait_scat(s)
                load(nxt, s)

        return carry

    lax.fori_loop(0, _NFULL // 3, trip, 0)
    for s in range(3):
        wait_scat(s)

    tv = v0.at[pl.ds(0, _TAIL)]
    pltpu.sync_copy(vals.at[pl.ds(e0 + _NFULL * _C, _TAIL)], tv)
    pltpu.sync_copy(tv, acc.at[idv.at[pl.ds(_NFULL * _C, _TAIL)]], add=True)

    plsc.subcore_barrier()
    pltpu.sync_copy(acc.at[pl.ds(r0, _TPW)], out.at[cid, pl.ds(r0, _TPW)])


_sc_scatter = pl.kernel(
    _sc_scatter_body,
    out_type=jax.ShapeDtypeStruct((2, _N, 32), _f32),
    mesh=_MESH,
    scratch_types=[
        pltpu.VMEM((_EPT,), jnp.int32),
        pltpu.VMEM((_C, 32), _f32),
        pltpu.VMEM((_C, 32), _f32),
        pltpu.VMEM((_C, 32), _f32),
        pltpu.VMEM_SHARED((_N, 32), _f32),
        pltpu.SemaphoreType.DMA,
        pltpu.SemaphoreType.DMA,
        pltpu.SemaphoreType.DMA,
        pltpu.SemaphoreType.DMA,
        pltpu.SemaphoreType.DMA,
        pltpu.SemaphoreType.DMA,
    ],
    compiler_params=_SC_PARAMS,
)


def _dot(a, b):
    return jnp.dot(a, b, preferred_element_type=_f32)


def _tc_nodeproj_body(x, wp, ps, pd):
    p = _dot(x[...], wp[...])
    ps[...] = p[:, :64]
    pd[...] = p[:, 64:]


def _tc_edge1_body(g, ea, w1e, b1, w2, b2, out):
    h = jnp.maximum(g[...] + _dot(ea[...], w1e[...]) + b1[...], 0.0)
    out[...] = _dot(h, w2[...]) + b2[...]


def _tc_node_mid_body(x, aggp, wx, wa, bih, bhh, wqx, wqh, h1o, qso, qdo):
    agg = aggp[0] + aggp[1]
    gi = _dot(x[...], wx[...]) + _dot(agg, wa[...]) + bih[...]
    b = bhh[...]
    r = jax.nn.sigmoid(gi[:, :64] + b[:, :64])
    z = jax.nn.sigmoid(gi[:, 64:128] + b[:, 64:128])
    nn_ = jnp.tanh(gi[:, 128:] + r * b[:, 128:])
    h1 = (1.0 - z) * nn_
    q = _dot(x[...], wqx[...]) + _dot(h1, wqh[...])
    h1o[...] = h1
    qso[...] = q[:, :64]
    qdo[...] = q[:, 64:]


def _tc_edge2pre_body(ea, e1, wea, we1, b1, out):
    out[...] = _dot(ea[...], wea[...]) + _dot(e1[...], we1[...]) + b1[...]


def _tc_edge2_body(g, t2, w2, b2, out):
    h = jnp.maximum(g[...] + t2[...], 0.0)
    out[...] = _dot(h, w2[...]) + b2[...]


def _tc_node_out_body(x, h1, aggp, wx, wh, wa, b1, w2, b2, out):
    agg = aggp[0] + aggp[1]
    h3 = jnp.maximum(
        _dot(x[...], wx[...]) + _dot(h1[...], wh[...]) + _dot(agg, wa[...])
        + b1[...], 0.0)
    out[...] = _dot(h3, w2[...]) + b2[...]


def _full(shape):
    return pl.BlockSpec(shape, lambda i: tuple(0 for _ in shape))


def _rows(blk, cols):
    return pl.BlockSpec((blk, cols), lambda i: (i, 0))


def _aggspec(blk):
    return pl.BlockSpec((2, blk, 32), lambda i: (0, i, 0))


_GRID_N = (_N // _BLK_N,)
_GRID_E = (_E // _BLK_E,)

_tc_nodeproj = pl.pallas_call(
    _tc_nodeproj_body,
    grid=_GRID_N,
    in_specs=[_rows(_BLK_N, 128), _full((128, 128))],
    out_specs=[_rows(_BLK_N, 64)] * 2,
    out_shape=[jax.ShapeDtypeStruct((_N, 64), _f32)] * 2,
)

_tc_edge1 = pl.pallas_call(
    _tc_edge1_body,
    grid=_GRID_E,
    in_specs=[_rows(_BLK_E, 64), _rows(_BLK_E, 16),
              _full((16, 64)), _full((1, 64)), _full((64, 32)), _full((1, 32))],
    out_specs=_rows(_BLK_E, 32),
    out_shape=jax.ShapeDtypeStruct((_E, 32), _f32),
)

_tc_node_mid = pl.pallas_call(
    _tc_node_mid_body,
    grid=_GRID_N,
    in_specs=[_rows(_BLK_N, 128), _aggspec(_BLK_N),
              _full((128, 192)), _full((32, 192)), _full((1, 192)),
              _full((1, 192)), _full((128, 128)), _full((64, 128))],
    out_specs=[_rows(_BLK_N, 64)] * 3,
    out_shape=[jax.ShapeDtypeStruct((_N, 64), _f32)] * 3,
)

_tc_edge2pre = pl.pallas_call(
    _tc_edge2pre_body,
    grid=_GRID_E,
    in_specs=[_rows(_BLK_E, 16), _rows(_BLK_E, 32),
              _full((16, 64)), _full((32, 64)), _full((1, 64))],
    out_specs=_rows(_BLK_E, 64),
    out_shape=jax.ShapeDtypeStruct((_E, 64), _f32),
)

_tc_edge2 = pl.pallas_call(
    _tc_edge2_body,
    grid=_GRID_E,
    in_specs=[_rows(_BLK_E, 64), _rows(_BLK_E, 64),
              _full((64, 32)), _full((1, 32))],
    out_specs=_rows(_BLK_E, 32),
    out_shape=jax.ShapeDtypeStruct((_E, 32), _f32),
)

_tc_node_out = pl.pallas_call(
    _tc_node_out_body,
    grid=_GRID_N,
    in_specs=[_rows(_BLK_N, 128), _rows(_BLK_N, 64), _aggspec(_BLK_N),
              _full((128, 64)), _full((64, 64)), _full((32, 64)),
              _full((1, 64)), _full((64, 128)), _full((1, 128))],
    out_specs=_rows(_BLK_N, 128),
    out_shape=jax.ShapeDtypeStruct((_N, 128), _f32),
)


def kernel(x, edge_index, edge_attr, params):
    p = params
    src = edge_index[0]
    dst = edge_index[1]

    w1 = p["gn1_e_W1"]
    wp = jnp.concatenate([w1[:128], w1[128:256]], axis=1)
    w1e = w1[256:]
    b11 = p["gn1_e_b1"].reshape(1, 64)
    w12 = p["gn1_e_W2"]
    b12 = p["gn1_e_b2"].reshape(1, 32)

    wih_t = p["gru_Wih"].T
    wx = wih_t[:128]
    wa = wih_t[128:]
    bih = p["gru_bih"].reshape(1, 192)
    bhh = p["gru_bhh"].reshape(1, 192)

    w2_ = p["gn2_e_W1"]
    wq = jnp.concatenate([w2_[:192], w2_[192:384]], axis=1)
    wqx = wq[:128]
    wqh = wq[128:]
    wea2 = w2_[384:400]
    we12 = w2_[400:432]
    b21 = p["gn2_e_b1"].reshape(1, 64)
    w22 = p["gn2_e_W2"]
    b22 = p["gn2_e_b2"].reshape(1, 32)

    wn1 = p["gn2_n_W1"]
    wnx = wn1[:128]
    wnh = wn1[128:192]
    wna = wn1[192:]
    bn1 = p["gn2_n_b1"].reshape(1, 64)
    wn2 = p["gn2_n_W2"]
    bn2 = p["gn2_n_b2"].reshape(1, 128)

    zz = jnp.zeros((_N, 32), _f32)

    ps, pd = _tc_nodeproj(x, wp)
    g1 = _sc_gather(ps, pd, src, dst)
    e1 = _tc_edge1(g1, edge_attr, w1e, b11, w12, b12)
    t2 = _tc_edge2pre(edge_attr, e1, wea2, we12, b21)
    agg1p = _sc_scatter(e1, dst, zz)
    h1, qs, qd = _tc_node_mid(x, agg1p, wx, wa, bih, bhh, wqx, wqh)
    g2 = _sc_gather(qs, qd, src, dst)
    e2 = _tc_edge2(g2, t2, w22, b22)
    agg2p = _sc_scatter(e2, dst, zz)
    out = _tc_node_out(x, h1, agg2p, wnx, wnh, wna, bn1, wn2, bn2)
    return out, h1[None]


# speedup vs baseline: 1.1840x; 1.0756x over previous
import functools

import jax
import jax.numpy as jnp
from jax import lax
from jax.experimental import pallas as pl
from jax.experimental.pallas import tpu as pltpu
from jax.experimental.pallas import tpu_sc as plsc

_N = 10000
_E = 320000
_BLK_N = 5000
_BLK_E = 8000
_C = 128
_NW = 32
_EPT = _E // _NW
_NFULL = _EPT // _C
_TAIL = _EPT - _NFULL * _C
_TPW = _N // 16

_f32 = jnp.float32
_MESH = plsc.VectorSubcoreMesh(core_axis_name="c", subcore_axis_name="s")
_SC_PARAMS = pltpu.CompilerParams(use_tc_tiling_on_sc=False)


def _add_rows(dst_v, a_v, b_v, nrows):
    def row(r, carry):
        for cc in range(4):
            sl = pl.ds(cc * 16, 16)
            dst_v[r, sl] = a_v[r, sl] + b_v[r, sl]
        return carry

    lax.fori_loop(0, nrows, row, 0)


def _sc_gather_body(ps, pd, src_i, dst_i, g,
                    isv, idv, rs0, rd0, rs1, rd1, g0, g1,
                    ss0, sd0, ss1, sd1, so0, so1):
    wid = lax.axis_index("s") * 2 + lax.axis_index("c")
    e0 = wid * _EPT
    pltpu.sync_copy(src_i.at[pl.ds(e0, _EPT)], isv)
    pltpu.sync_copy(dst_i.at[pl.ds(e0, _EPT)], idv)

    def issue(c, rs, rd, sem_s, sem_d):
        sl = pl.ds(c * _C, _C)
        pltpu.async_copy(ps.at[isv.at[sl]], rs, sem_s)
        pltpu.async_copy(pd.at[idv.at[sl]], rd, sem_d)

    def waitg(rs, rd, sem_s, sem_d):
        pltpu.make_async_copy(ps.at[isv.at[pl.ds(0, _C)]], rs, sem_s).wait()
        pltpu.make_async_copy(pd.at[idv.at[pl.ds(0, _C)]], rd, sem_d).wait()

    def store(c, gb, sem_o):
        pltpu.async_copy(gb, g.at[pl.ds(e0 + c * _C, _C)], sem_o)

    def drain(gb, sem_o):
        pltpu.make_async_copy(gb, g.at[pl.ds(e0, _C)], sem_o).wait()

    issue(0, rs0, rd0, ss0, sd0)

    def pair(t, carry):
        a = 2 * t
        issue(a + 1, rs1, rd1, ss1, sd1)
        waitg(rs0, rd0, ss0, sd0)

        @pl.when(t > 0)
        def _():
            drain(g0, so0)

        _add_rows(g0, rs0, rd0, _C)
        store(a, g0, so0)

        @pl.when(t < _NFULL // 2 - 1)
        def _():
            issue(a + 2, rs0, rd0, ss0, sd0)

        waitg(rs1, rd1, ss1, sd1)

        @pl.when(t > 0)
        def _():
            drain(g1, so1)

        _add_rows(g1, rs1, rd1, _C)
        store(a + 1, g1, so1)
        return carry

    lax.fori_loop(0, _NFULL // 2, pair, 0)
    drain(g0, so0)
    drain(g1, so1)

    tsl = pl.ds(_NFULL * _C, _TAIL)
    rs_t = rs0.at[pl.ds(0, _TAIL)]
    rd_t = rd0.at[pl.ds(0, _TAIL)]
    pltpu.async_copy(ps.at[isv.at[tsl]], rs_t, ss0)
    pltpu.async_copy(pd.at[idv.at[tsl]], rd_t, sd0)
    pltpu.make_async_copy(ps.at[isv.at[tsl]], rs_t, ss0).wait()
    pltpu.make_async_copy(pd.at[idv.at[tsl]], rd_t, sd0).wait()
    _add_rows(g0, rs0, rd0, _TAIL)
    pltpu.sync_copy(g0.at[pl.ds(0, _TAIL)],
                    g.at[pl.ds(e0 + _NFULL * _C, _TAIL)])


_sc_gather = pl.kernel(
    _sc_gather_body,
    out_type=jax.ShapeDtypeStruct((_E, 64), _f32),
    mesh=_MESH,
    scratch_types=[
        pltpu.VMEM((_EPT,), jnp.int32),
        pltpu.VMEM((_EPT,), jnp.int32),
        pltpu.VMEM((_C, 64), _f32),
        pltpu.VMEM((_C, 64), _f32),
        pltpu.VMEM((_C, 64), _f32),
        pltpu.VMEM((_C, 64), _f32),
        pltpu.VMEM((_C, 64), _f32),
        pltpu.VMEM((_C, 64), _f32),
        pltpu.SemaphoreType.DMA,
        pltpu.SemaphoreType.DMA,
        pltpu.SemaphoreType.DMA,
        pltpu.SemaphoreType.DMA,
        pltpu.SemaphoreType.DMA,
        pltpu.SemaphoreType.DMA,
    ],
    compiler_params=_SC_PARAMS,
)


def _sc_scatter_body(vals, dst_i, zz, out,
                     idv, v0, v1, v2, acc,
                     sv0, sv1, sv2, sa0, sa1, sa2):
    cid = lax.axis_index("c")
    sid = lax.axis_index("s")
    wid = sid * 2 + cid
    e0 = wid * _EPT
    r0 = sid * _TPW
    pltpu.sync_copy(zz.at[pl.ds(r0, _TPW)], acc.at[pl.ds(r0, _TPW)])
    pltpu.sync_copy(dst_i.at[pl.ds(e0, _EPT)], idv)
    plsc.subcore_barrier()

    vb = (v0, v1, v2)
    sv = (sv0, sv1, sv2)
    sa = (sa0, sa1, sa2)

    def load(c, s):
        pltpu.async_copy(vals.at[pl.ds(e0 + c * _C, _C)], vb[s], sv[s])

    def wait_load(s):
        pltpu.make_async_copy(vals.at[pl.ds(e0, _C)], vb[s], sv[s]).wait()

    def scat(c, s):
        pltpu.async_copy(vb[s], acc.at[idv.at[pl.ds(c * _C, _C)]], sa[s],
                         add=True)

    def wait_scat(s):
        pltpu.make_async_copy(vb[s], acc.at[idv.at[pl.ds(0, _C)]],
                              sa[s]).wait()

    for s in range(3):
        load(s, s)

    def trip(t, carry):
        c = 3 * t
        for s in range(3):
            wait_load(s)
            scat(c + s, s)
        for s in range(3):
            nxt = c + 3 + s

            @pl.when(nxt < _NFULL)
            def _():
                wait_scat(s)
                load(nxt, s)

        return carry

    lax.fori_loop(0, _NFULL // 3, trip, 0)
    for s in range(3):
        wait_scat(s)

    tv = v0.at[pl.ds(0, _TAIL)]
    pltpu.sync_copy(vals.at[pl.ds(e0 + _NFULL * _C, _TAIL)], tv)
    pltpu.sync_copy(tv, acc.at[idv.at[pl.ds(_NFULL * _C, _TAIL)]], add=True)

    plsc.subcore_barrier()
    pltpu.sync_copy(acc.at[pl.ds(r0, _TPW)], out.at[cid, pl.ds(r0, _TPW)])


_sc_scatter = pl.kernel(
    _sc_scatter_body,
    out_type=jax.ShapeDtypeStruct((2, _N, 32), _f32),
    mesh=_MESH,
    scratch_types=[
        pltpu.VMEM((_EPT,), jnp.int32),
        pltpu.VMEM((_C, 32), _f32),
        pltpu.VMEM((_C, 32), _f32),
        pltpu.VMEM((_C, 32), _f32),
        pltpu.VMEM_SHARED((_N, 32), _f32),
        pltpu.SemaphoreType.DMA,
        pltpu.SemaphoreType.DMA,
        pltpu.SemaphoreType.DMA,
        pltpu.SemaphoreType.DMA,
        pltpu.SemaphoreType.DMA,
        pltpu.SemaphoreType.DMA,
    ],
    compiler_params=_SC_PARAMS,
)


def _dot(a, b):
    return jnp.dot(a, b, preferred_element_type=_f32)


def _tc_nodeproj_body(x, wp, ps, pd):
    p = _dot(x[...], wp[...])
    ps[...] = p[:, :64]
    pd[...] = p[:, 64:]


def _tc_edge1_body(g, ea, w1e, b1, w2, b2, out):
    h = jnp.maximum(g[...] + _dot(ea[...], w1e[...]) + b1[...], 0.0)
    out[...] = _dot(h, w2[...]) + b2[...]


def _tc_node_mid_body(x, aggp, wx, wa, bih, bhh, wqx, wqh, h1o, qso, qdo):
    agg = aggp[0] + aggp[1]
    gi = _dot(x[...], wx[...]) + _dot(agg, wa[...]) + bih[...]
    b = bhh[...]
    r = jax.nn.sigmoid(gi[:, :64] + b[:, :64])
    z = jax.nn.sigmoid(gi[:, 64:128] + b[:, 64:128])
    nn_ = jnp.tanh(gi[:, 128:] + r * b[:, 128:])
    h1 = (1.0 - z) * nn_
    q = _dot(x[...], wqx[...]) + _dot(h1, wqh[...])
    h1o[...] = h1
    qso[...] = q[:, :64]
    qdo[...] = q[:, 64:]


def _tc_edge2_body(g, ea, e1, wea, we1, b1, w2, b2, out):
    h = jnp.maximum(
        g[...] + _dot(ea[...], wea[...]) + _dot(e1[...], we1[...])
        + b1[...], 0.0)
    out[...] = _dot(h, w2[...]) + b2[...]


def _tc_node_out_body(x, h1, aggp, wx, wh, wa, b1, w2, b2, out):
    agg = aggp[0] + aggp[1]
    h3 = jnp.maximum(
        _dot(x[...], wx[...]) + _dot(h1[...], wh[...]) + _dot(agg, wa[...])
        + b1[...], 0.0)
    out[...] = _dot(h3, w2[...]) + b2[...]


def _full(shape):
    return pl.BlockSpec(shape, lambda i: tuple(0 for _ in shape))


def _rows(blk, cols):
    return pl.BlockSpec((blk, cols), lambda i: (i, 0))


def _aggspec(blk):
    return pl.BlockSpec((2, blk, 32), lambda i: (0, i, 0))


_GRID_N = (_N // _BLK_N,)
_GRID_E = (_E // _BLK_E,)

_tc_nodeproj = pl.pallas_call(
    _tc_nodeproj_body,
    grid=_GRID_N,
    in_specs=[_rows(_BLK_N, 128), _full((128, 128))],
    out_specs=[_rows(_BLK_N, 64)] * 2,
    out_shape=[jax.ShapeDtypeStruct((_N, 64), _f32)] * 2,
)

_tc_edge1 = pl.pallas_call(
    _tc_edge1_body,
    grid=_GRID_E,
    in_specs=[_rows(_BLK_E, 64), _rows(_BLK_E, 16),
              _full((16, 64)), _full((1, 64)), _full((64, 32)), _full((1, 32))],
    out_specs=_rows(_BLK_E, 32),
    out_shape=jax.ShapeDtypeStruct((_E, 32), _f32),
)

_tc_node_mid = pl.pallas_call(
    _tc_node_mid_body,
    grid=_GRID_N,
    in_specs=[_rows(_BLK_N, 128), _aggspec(_BLK_N),
              _full((128, 192)), _full((32, 192)), _full((1, 192)),
              _full((1, 192)), _full((128, 128)), _full((64, 128))],
    out_specs=[_rows(_BLK_N, 64)] * 3,
    out_shape=[jax.ShapeDtypeStruct((_N, 64), _f32)] * 3,
)

_tc_edge2 = pl.pallas_call(
    _tc_edge2_body,
    grid=_GRID_E,
    in_specs=[_rows(_BLK_E, 64), _rows(_BLK_E, 16),
              _rows(_BLK_E, 32), _full((16, 64)), _full((32, 64)),
              _full((1, 64)), _full((64, 32)), _full((1, 32))],
    out_specs=_rows(_BLK_E, 32),
    out_shape=jax.ShapeDtypeStruct((_E, 32), _f32),
)

_tc_node_out = pl.pallas_call(
    _tc_node_out_body,
    grid=_GRID_N,
    in_specs=[_rows(_BLK_N, 128), _rows(_BLK_N, 64), _aggspec(_BLK_N),
              _full((128, 64)), _full((64, 64)), _full((32, 64)),
              _full((1, 64)), _full((64, 128)), _full((1, 128))],
    out_specs=_rows(_BLK_N, 128),
    out_shape=jax.ShapeDtypeStruct((_N, 128), _f32),
)


def kernel(x, edge_index, edge_attr, params):
    p = params
    src = edge_index[0]
    dst = edge_index[1]

    w1 = p["gn1_e_W1"]
    wp = jnp.concatenate([w1[:128], w1[128:256]], axis=1)
    w1e = w1[256:]
    b11 = p["gn1_e_b1"].reshape(1, 64)
    w12 = p["gn1_e_W2"]
    b12 = p["gn1_e_b2"].reshape(1, 32)

    wih_t = p["gru_Wih"].T
    wx = wih_t[:128]
    wa = wih_t[128:]
    bih = p["gru_bih"].reshape(1, 192)
    bhh = p["gru_bhh"].reshape(1, 192)

    w2_ = p["gn2_e_W1"]
    wq = jnp.concatenate([w2_[:192], w2_[192:384]], axis=1)
    wqx = wq[:128]
    wqh = wq[128:]
    wea2 = w2_[384:400]
    we12 = w2_[400:432]
    b21 = p["gn2_e_b1"].reshape(1, 64)
    w22 = p["gn2_e_W2"]
    b22 = p["gn2_e_b2"].reshape(1, 32)

    wn1 = p["gn2_n_W1"]
    wnx = wn1[:128]
    wnh = wn1[128:192]
    wna = wn1[192:]
    bn1 = p["gn2_n_b1"].reshape(1, 64)
    wn2 = p["gn2_n_W2"]
    bn2 = p["gn2_n_b2"].reshape(1, 128)

    zz = jnp.zeros((_N, 32), _f32)

    ps, pd = _tc_nodeproj(x, wp)
    g1 = _sc_gather(ps, pd, src, dst)
    e1 = _tc_edge1(g1, edge_attr, w1e, b11, w12, b12)
    agg1p = _sc_scatter(e1, dst, zz)
    h1, qs, qd = _tc_node_mid(x, agg1p, wx, wa, bih, bhh, wqx, wqh)
    g2 = _sc_gather(qs, qd, src, dst)
    e2 = _tc_edge2(g2, edge_attr, e1, wea2, we12, b21, w22, b22)
    agg2p = _sc_scatter(e2, dst, zz)
    out = _tc_node_out(x, h1, agg2p, wnx, wnh, wna, bn1, wn2, bn2)
    return out, h1[None]
